# Initial kernel scaffold; baseline (speedup 1.0000x reference)
import jax
import jax.numpy as jnp
from jax.experimental import pallas as pl


def kernel(net, inp, corr, ii, jj, kk, params):
    raise NotImplementedError("write your pallas kernel here")


# speedup vs baseline: 2.3660x; 2.3660x over previous
import functools

import jax
import jax.numpy as jnp
from jax import lax
from jax.experimental import pallas as pl
from jax.experimental.pallas import tpu as pltpu
from jax.experimental.pallas import tpu_sc as plsc

_DIM = 384
_E = 8192
_BLK = 1024
_NBLK = _E // _BLK
_EPAD = _E + _BLK
_CORR = 2 * 49 * 5 * 5
_SKK = 256
_SIJ = 1024
_NF = 32
_NW = 32
_BPW = _E // _NW


def _lnorm(x, g, b):
    m = jnp.mean(x, axis=-1, keepdims=True)
    xc = x - m
    v = jnp.mean(xc * xc, axis=-1, keepdims=True)
    return xc * lax.rsqrt(v + 1e-3) * g + b


def _blk_spec():
    return pl.BlockSpec((_BLK, _DIM), lambda i: (jnp.minimum(i, _NBLK - 1), 0))


def _full(shape):
    return pl.BlockSpec(shape, lambda i: (0,) * len(shape))


def _seg_spec():
    return pl.BlockSpec((1, 1, _BLK), lambda i: (i, 0, 0))


_CP = pltpu.CompilerParams(dimension_semantics=("arbitrary",))


def _k1_body(corr_ref, net_ref, inp_ref, w1, b1, w2, b2, lng, lnb, w3, b3,
             ng, nb, out_ref):
    i = pl.program_id(0)

    @pl.when(i < _NBLK)
    def _():
        c = jnp.dot(corr_ref[...], w1[...], preferred_element_type=jnp.float32)
        c = jnp.maximum(c + b1[...], 0.0)
        c = jnp.dot(c, w2[...], preferred_element_type=jnp.float32) + b2[...]
        c = jnp.maximum(_lnorm(c, lng[...], lnb[...]), 0.0)
        c = jnp.dot(c, w3[...], preferred_element_type=jnp.float32) + b3[...]
        x = net_ref[...] + inp_ref[...] + c
        out_ref[...] = _lnorm(x, ng[...], nb[...])

    @pl.when(i == _NBLK)
    def _():
        out_ref[...] = jnp.zeros_like(out_ref)


def _k1_call(corr2d, net2d, inp2d, p):
    return pl.pallas_call(
        _k1_body,
        grid=(_NBLK + 1,),
        in_specs=[
            pl.BlockSpec((_BLK, _CORR), lambda i: (jnp.minimum(i, _NBLK - 1), 0)),
            _blk_spec(), _blk_spec(),
            _full((_CORR, _DIM)), _full((1, _DIM)),
            _full((_DIM, _DIM)), _full((1, _DIM)),
            _full((1, _DIM)), _full((1, _DIM)),
            _full((_DIM, _DIM)), _full((1, _DIM)),
            _full((1, _DIM)), _full((1, _DIM)),
        ],
        out_specs=pl.BlockSpec((_BLK, _DIM), lambda i: (i, 0)),
        out_shape=jax.ShapeDtypeStruct((_EPAD, _DIM), jnp.float32),
        compiler_params=_CP,
    )(corr2d, net2d, inp2d,
      p['corr1_w'], p['corr1_b'].reshape(1, -1),
      p['corr2_w'], p['corr2_b'].reshape(1, -1),
      p['corr_ln_g'].reshape(1, -1), p['corr_ln_b'].reshape(1, -1),
      p['corr3_w'], p['corr3_b'].reshape(1, -1),
      p['norm_g'].reshape(1, -1), p['norm_b'].reshape(1, -1))


@functools.cache
def _sc_gather_kernel():
    @functools.partial(
        pl.kernel,
        out_type=jax.ShapeDtypeStruct((_E, _DIM), jnp.float32),
        mesh=plsc.VectorSubcoreMesh(core_axis_name="c", subcore_axis_name="s"),
        scratch_types=[
            pltpu.VMEM((2, 128), jnp.int32),
            pltpu.VMEM((_BPW, _DIM), jnp.float32),
            pltpu.SemaphoreType.DMA,
            pltpu.SemaphoreType.DMA,
        ],
    )
    def gather(table_hbm, idx_hbm, out_hbm, idx_v, rows_v, sem0, sem1):
        wid = lax.axis_index("s") * 2 + lax.axis_index("c")
        pltpu.sync_copy(idx_hbm.at[pl.ds(wid * 2, 2)], idx_v)
        c0 = pltpu.async_copy(table_hbm.at[idx_v.at[0]],
                              rows_v.at[pl.ds(0, 128)], sem0)
        c1 = pltpu.async_copy(table_hbm.at[idx_v.at[1]],
                              rows_v.at[pl.ds(128, 128)], sem1)
        c0.wait()
        c1.wait()
        pltpu.sync_copy(rows_v, out_hbm.at[pl.ds(wid * _BPW, _BPW)])
    return gather


def _sc_gather(table, idx):
    return _sc_gather_kernel()(table, idx)


def _k2_body(net_ref, h_ref, wa, ba, wb, bb, out_ref):
    i = pl.program_id(0)

    @pl.when(i < _NBLK)
    def _():
        t = jnp.maximum(jnp.dot(h_ref[...], wa[...],
                                preferred_element_type=jnp.float32) + ba[...], 0.0)
        out_ref[...] = net_ref[...] + jnp.dot(
            t, wb[...], preferred_element_type=jnp.float32) + bb[...]

    @pl.when(i == _NBLK)
    def _():
        out_ref[...] = jnp.zeros_like(out_ref)


def _k2_call(netz, h, p, pre):
    return pl.pallas_call(
        _k2_body,
        grid=(_NBLK + 1,),
        in_specs=[_blk_spec(), _blk_spec(),
                  _full((_DIM, _DIM)), _full((1, _DIM)),
                  _full((_DIM, _DIM)), _full((1, _DIM))],
        out_specs=pl.BlockSpec((_BLK, _DIM), lambda i: (i, 0)),
        out_shape=jax.ShapeDtypeStruct((_EPAD, _DIM), jnp.float32),
        compiler_params=_CP,
    )(netz, h,
      p[pre + 'a_w'], p[pre + 'a_b'].reshape(1, -1),
      p[pre + 'b_w'], p[pre + 'b_b'].reshape(1, -1))


def _k3_body(net_ref, h_ref, c2aw, c2ab, c2bw, c2bb, fw, fb, gw, gb,
             net3_ref, f_ref, g_ref, gmax_ref, mx):
    i = pl.program_id(0)
    t = jnp.maximum(jnp.dot(h_ref[...], c2aw[...],
                            preferred_element_type=jnp.float32) + c2ab[...], 0.0)
    n3 = net_ref[...] + jnp.dot(t, c2bw[...],
                                preferred_element_type=jnp.float32) + c2bb[...]
    net3_ref[...] = n3
    f = jnp.dot(n3, fw[...], preferred_element_type=jnp.float32) + fb[...]
    g = jnp.dot(n3, gw[...], preferred_element_type=jnp.float32) + gb[...]
    f_ref[...] = f
    g_ref[...] = g
    bm = jnp.max(g, axis=0, keepdims=True)

    @pl.when(i == 0)
    def _():
        mx[...] = bm

    @pl.when(i > 0)
    def _():
        mx[...] = jnp.maximum(mx[...], bm)

    @pl.when(i == _NBLK - 1)
    def _():
        gmax_ref[...] = mx[...]


def _k3_call(netz, h, p):
    return pl.pallas_call(
        _k3_body,
        grid=(_NBLK,),
        in_specs=[_blk_spec(), _blk_spec(),
                  _full((_DIM, _DIM)), _full((1, _DIM)),
                  _full((_DIM, _DIM)), _full((1, _DIM)),
                  _full((_DIM, _DIM)), _full((1, _DIM)),
                  _full((_DIM, _DIM)), _full((1, _DIM))],
        out_specs=[pl.BlockSpec((_BLK, _DIM), lambda i: (i, 0)),
                   pl.BlockSpec((_BLK, _DIM), lambda i: (i, 0)),
                   pl.BlockSpec((_BLK, _DIM), lambda i: (i, 0)),
                   _full((1, _DIM))],
        out_shape=[jax.ShapeDtypeStruct((_E, _DIM), jnp.float32),
                   jax.ShapeDtypeStruct((_E, _DIM), jnp.float32),
                   jax.ShapeDtypeStruct((_E, _DIM), jnp.float32),
                   jax.ShapeDtypeStruct((1, _DIM), jnp.float32)],
        scratch_shapes=[pltpu.VMEM((1, _DIM), jnp.float32)],
        compiler_params=_CP,
    )(netz, h,
      p['c2a_w'], p['c2a_b'].reshape(1, -1),
      p['c2b_w'], p['c2b_b'].reshape(1, -1),
      p['agg_kk_f_w'], p['agg_kk_f_b'].reshape(1, -1),
      p['agg_kk_g_w'], p['agg_kk_g_b'].reshape(1, -1))


def _agg_body(S):
    def body(f_ref, g_ref, gmax_ref, seg_ref, wh_ref, yh_ref, num, den):
        i = pl.program_id(0)
        w = jnp.exp(g_ref[...] - gmax_ref[...])
        M = (lax.broadcasted_iota(jnp.int32, (S, _BLK), 0)
             == seg_ref[0]).astype(jnp.float32)
        nu = jnp.dot(M, f_ref[...] * w, preferred_element_type=jnp.float32)
        de = jnp.dot(M, w, preferred_element_type=jnp.float32)

        @pl.when(i == 0)
        def _():
            num[...] = nu
            den[...] = de

        @pl.when(i > 0)
        def _():
            num[...] += nu
            den[...] += de

        @pl.when(i == _NBLK - 1)
        def _():
            d = den[...]
            y = num[...] / jnp.where(d > 0.0, d, 1.0)
            yh_ref[...] = jnp.dot(y, wh_ref[...],
                                  preferred_element_type=jnp.float32)
    return body


def _agg_call(S, f, g, gmax, seg3, wh):
    return pl.pallas_call(
        _agg_body(S),
        grid=(_NBLK,),
        in_specs=[_blk_spec(), _blk_spec(), _full((1, _DIM)), _seg_spec(),
                  _full((_DIM, _DIM))],
        out_specs=_full((S, _DIM)),
        out_shape=jax.ShapeDtypeStruct((S, _DIM), jnp.float32),
        scratch_shapes=[pltpu.VMEM((S, _DIM), jnp.float32),
                        pltpu.VMEM((S, _DIM), jnp.float32)],
        compiler_params=_CP,
    )(f, g, gmax, seg3, wh)


def _k5a_body(net3_ref, yh_ref, seg_ref, bh, fw, fb, gw, gb,
              net4_ref, f_ref, g_ref, gmax_ref, mx):
    i = pl.program_id(0)
    M = (lax.broadcasted_iota(jnp.int32, (_SKK, _BLK), 0)
         == seg_ref[0]).astype(jnp.float32)
    n4 = net3_ref[...] + lax.dot_general(
        M, yh_ref[...], (((0,), (0,)), ((), ())),
        preferred_element_type=jnp.float32) + bh[...]
    net4_ref[...] = n4
    f = jnp.dot(n4, fw[...], preferred_element_type=jnp.float32) + fb[...]
    g = jnp.dot(n4, gw[...], preferred_element_type=jnp.float32) + gb[...]
    f_ref[...] = f
    g_ref[...] = g
    bm = jnp.max(g, axis=0, keepdims=True)

    @pl.when(i == 0)
    def _():
        mx[...] = bm

    @pl.when(i > 0)
    def _():
        mx[...] = jnp.maximum(mx[...], bm)

    @pl.when(i == _NBLK - 1)
    def _():
        gmax_ref[...] = mx[...]


def _k5a_call(net3, yh1, kk3, p):
    return pl.pallas_call(
        _k5a_body,
        grid=(_NBLK,),
        in_specs=[_blk_spec(), _full((_SKK, _DIM)), _seg_spec(),
                  _full((1, _DIM)),
                  _full((_DIM, _DIM)), _full((1, _DIM)),
                  _full((_DIM, _DIM)), _full((1, _DIM))],
        out_specs=[pl.BlockSpec((_BLK, _DIM), lambda i: (i, 0)),
                   pl.BlockSpec((_BLK, _DIM), lambda i: (i, 0)),
                   pl.BlockSpec((_BLK, _DIM), lambda i: (i, 0)),
                   _full((1, _DIM))],
        out_shape=[jax.ShapeDtypeStruct((_E, _DIM), jnp.float32),
                   jax.ShapeDtypeStruct((_E, _DIM), jnp.float32),
                   jax.ShapeDtypeStruct((_E, _DIM), jnp.float32),
                   jax.ShapeDtypeStruct((1, _DIM), jnp.float32)],
        scratch_shapes=[pltpu.VMEM((1, _DIM), jnp.float32)],
        compiler_params=_CP,
    )(net3, yh1, kk3,
      p['agg_kk_h_b'].reshape(1, -1),
      p['agg_ij_f_w'], p['agg_ij_f_b'].reshape(1, -1),
      p['agg_ij_g_w'], p['agg_ij_g_b'].reshape(1, -1))


def _k6_body(net4_ref, yh_ref, seg_ref, bh, l1g, l1b, g1w, g1b, r1aw, r1ab,
             r1bw, r1bb, l2g, l2b, g2w, g2b, r2aw, r2ab, r2bw, r2bb,
             hw, hb, net_ref, dw_ref):
    M = (lax.broadcasted_iota(jnp.int32, (_SIJ, _BLK), 0)
         == seg_ref[0]).astype(jnp.float32)
    x = net4_ref[...] + lax.dot_general(
        M, yh_ref[...], (((0,), (0,)), ((), ())),
        preferred_element_type=jnp.float32) + bh[...]
    x = _lnorm(x, l1g[...], l1b[...])
    gate = jax.nn.sigmoid(jnp.dot(x, g1w[...],
                                  preferred_element_type=jnp.float32) + g1b[...])
    res = jnp.maximum(jnp.dot(x, r1aw[...],
                              preferred_element_type=jnp.float32) + r1ab[...], 0.0)
    res = jnp.dot(res, r1bw[...], preferred_element_type=jnp.float32) + r1bb[...]
    x = x + gate * res
    x = _lnorm(x, l2g[...], l2b[...])
    gate = jax.nn.sigmoid(jnp.dot(x, g2w[...],
                                  preferred_element_type=jnp.float32) + g2b[...])
    res = jnp.maximum(jnp.dot(x, r2aw[...],
                              preferred_element_type=jnp.float32) + r2ab[...], 0.0)
    res = jnp.dot(res, r2bw[...], preferred_element_type=jnp.float32) + r2bb[...]
    x = x + gate * res
    net_ref[...] = x
    r = jnp.maximum(x, 0.0)
    hd = jnp.dot(r, hw[...], preferred_element_type=jnp.float32) + hb[...]
    lane = lax.broadcasted_iota(jnp.int32, hd.shape, 1)
    dw_ref[...] = jnp.where(lane >= 128, jax.nn.sigmoid(hd), hd)


def _k6_call(net4, yh2, sij3, p, hw, hb):
    return pl.pallas_call(
        _k6_body,
        grid=(_NBLK,),
        in_specs=[_blk_spec(), _full((_SIJ, _DIM)), _seg_spec(),
                  _full((1, _DIM)),
                  _full((1, _DIM)), _full((1, _DIM)),
                  _full((_DIM, _DIM)), _full((1, _DIM)),
                  _full((_DIM, _DIM)), _full((1, _DIM)),
                  _full((_DIM, _DIM)), _full((1, _DIM)),
                  _full((1, _DIM)), _full((1, _DIM)),
                  _full((_DIM, _DIM)), _full((1, _DIM)),
                  _full((_DIM, _DIM)), _full((1, _DIM)),
                  _full((_DIM, _DIM)), _full((1, _DIM)),
                  _full((_DIM, 256)), _full((1, 256))],
        out_specs=[pl.BlockSpec((_BLK, _DIM), lambda i: (i, 0)),
                   pl.BlockSpec((_BLK, 256), lambda i: (i, 0))],
        out_shape=[jax.ShapeDtypeStruct((_E, _DIM), jnp.float32),
                   jax.ShapeDtypeStruct((_E, 256), jnp.float32)],
        compiler_params=_CP,
    )(net4, yh2, sij3,
      p['agg_ij_h_b'].reshape(1, -1),
      p['gru_ln1_g'].reshape(1, -1), p['gru_ln1_b'].reshape(1, -1),
      p['gru_gr1_gate_w'], p['gru_gr1_gate_b'].reshape(1, -1),
      p['gru_gr1_res1_w'], p['gru_gr1_res1_b'].reshape(1, -1),
      p['gru_gr1_res2_w'], p['gru_gr1_res2_b'].reshape(1, -1),
      p['gru_ln2_g'].reshape(1, -1), p['gru_ln2_b'].reshape(1, -1),
      p['gru_gr2_gate_w'], p['gru_gr2_gate_b'].reshape(1, -1),
      p['gru_gr2_res1_w'], p['gru_gr2_res1_b'].reshape(1, -1),
      p['gru_gr2_res2_w'], p['gru_gr2_res2_b'].reshape(1, -1),
      hw, hb)


def kernel(net, inp, corr, ii, jj, kk, params):
    p = params
    net2d, inp2d, corr2d = net[0], inp[0], corr[0]

    table = -jnp.ones((_SKK, _NF + 2), dtype=jnp.int32)
    table = table.at[kk, jj + 1].set(jnp.arange(_E, dtype=jnp.int32))
    ix = table[kk, jj]
    jx = table[kk, jj + 2]
    ixz = jnp.where(ix >= 0, ix, _E).reshape(_NW * 2, 128)
    jxz = jnp.where(jx >= 0, jx, _E).reshape(_NW * 2, 128)
    kk3 = kk.reshape(_NBLK, 1, _BLK)
    sij3 = (ii * _NF + jj).reshape(_NBLK, 1, _BLK)

    hw = jnp.zeros((_DIM, 256), jnp.float32)
    hw = hw.at[:, 0:2].set(p['d_w']).at[:, 128:130].set(p['w_w'])
    hb = jnp.zeros((1, 256), jnp.float32)
    hb = hb.at[0, 0:2].set(p['d_b']).at[0, 128:130].set(p['w_b'])

    net1z = _k1_call(corr2d, net2d, inp2d, p)
    h1 = _sc_gather(net1z, ixz)
    net2z = _k2_call(net1z, h1, p, 'c1')
    h2 = _sc_gather(net2z, jxz)
    net3, fkk, gkk, gmax1 = _k3_call(net2z, h2, p)
    yh1 = _agg_call(_SKK, fkk, gkk, gmax1, kk3, p['agg_kk_h_w'])
    net4, fij, gij, gmax2 = _k5a_call(net3, yh1, kk3, p)
    yh2 = _agg_call(_SIJ, fij, gij, gmax2, sij3, p['agg_ij_h_w'])
    net6, dw = _k6_call(net4, yh2, sij3, p, hw, hb)

    return net6[None], dw[:, 0:2][None], dw[:, 128:130][None]
